# baseline (speedup 1.0000x reference)
import functools

import jax
import jax.numpy as jnp
from jax import lax
from jax.experimental import pallas as pl
from jax.experimental.pallas import tpu as pltpu
import jax.experimental.pallas.tpu_sc as plsc

N_NODES = 10000
MAX_DEG = 16
D_FEAT = 256
BATCH = 1024
S1 = 25
S2 = 10
HID = 128
NUM_CLASSES = 40

NW = 32
SEEDS_W = BATCH // NW
G_W = SEEDS_W * S1
E_W = G_W * S2
CH = 80
N_CH1 = G_W // CH
N_CH2 = E_W // CH
GPC = CH // S2
NBUF = 5

_PREC = lax.Precision.DEFAULT


def _proj_body(x_ref, w1n_ref, w1x_ref, p_ref, q_ref):
    x = x_ref[...]
    p_ref[...] = jnp.dot(x, w1n_ref[...], preferred_element_type=jnp.float32,
                         precision=_PREC)
    q_ref[...] = jnp.dot(x, w1x_ref[...], preferred_element_type=jnp.float32,
                         precision=_PREC)


def _project_tables(features, W1n, W1x):
    blk = 1000
    return pl.pallas_call(
        _proj_body,
        grid=(N_NODES // blk,),
        in_specs=[
            pl.BlockSpec((blk, D_FEAT), lambda i: (i, 0)),
            pl.BlockSpec((D_FEAT, HID), lambda i: (0, 0)),
            pl.BlockSpec((D_FEAT, HID), lambda i: (0, 0)),
        ],
        out_specs=[
            pl.BlockSpec((blk, HID), lambda i: (i, 0)),
            pl.BlockSpec((blk, HID), lambda i: (i, 0)),
        ],
        out_shape=[
            jax.ShapeDtypeStruct((N_NODES, HID), jnp.float32),
            jax.ShapeDtypeStruct((N_NODES, HID), jnp.float32),
        ],
    )(features, W1n, W1x)


def _sampler_body(ids_h, adjp_h, sel1_h, sel2_h,
                  s1_h, s2_h,
                  ids_v, sel1_v, s1_v, adjseed_v, sel2_v, s2_v,
                  abuf0, abuf1, sem, sA0, sA1):
    abuf, sA = (abuf0, abuf1), (sA0, sA1)
    w = lax.axis_index("c") * 16 + lax.axis_index("s")
    seed0 = w * SEEDS_W
    g0 = w * G_W
    e0 = w * E_W

    pltpu.sync_copy(sel2_h.at[pl.ds(e0, E_W)], sel2_v)
    pltpu.sync_copy(ids_h.at[pl.ds(seed0, SEEDS_W)], ids_v)
    pltpu.sync_copy(sel1_h.at[pl.ds(g0, G_W)], sel1_v)
    pltpu.async_copy(adjp_h.at[ids_v], adjseed_v, sem).wait()

    def s1_body(k, carry):
        sel = sel1_v[pl.ds(k * 16, 16)]
        row = jnp.right_shift(sel, 7)
        col = jnp.bitwise_and(sel, 127)
        s1_v[pl.ds(k * 16, 16)] = plsc.load_gather(adjseed_v, [row, col])
        return carry
    lax.fori_loop(0, G_W // 16, s1_body, 0)
    pltpu.sync_copy(s1_v, s1_h.at[pl.ds(g0, G_W)])

    def adj_gather(c, x):
        return pltpu.make_async_copy(
            adjp_h.at[s1_v.at[pl.ds(c * CH, CH)]], abuf[x], sA[x])

    def selT(c, x):
        adj_gather(c, x).wait()
        def sel_body(k, carry):
            sel = sel2_v[pl.ds(c * (CH * S2) + k * 16, 16)]
            row = jnp.right_shift(sel, 7)
            col = jnp.bitwise_and(sel, 127)
            s2_v[pl.ds(c * (CH * S2) + k * 16, 16)] = (
                plsc.load_gather(abuf[x], [row, col]))
            return carry
        lax.fori_loop(0, CH * S2 // 16, sel_body, 0)

        @pl.when(c + 2 < N_CH1)
        def _():
            adj_gather(c + 2, x).start()

    adj_gather(0, 0).start()
    adj_gather(1, 1).start()
    def passT(k, carry):
        selT(2 * k, 0)
        selT(2 * k + 1, 1)
        return carry
    lax.fori_loop(0, N_CH1 // 2, passT, 0)
    pltpu.sync_copy(s2_v, s2_h.at[pl.ds(e0, E_W)])


def _sampler(ids, adjp, sel1, sel2):
    mesh = plsc.VectorSubcoreMesh(core_axis_name="c", subcore_axis_name="s",
                                  num_cores=2, num_subcores=16)
    f = functools.partial(
        pl.kernel, _sampler_body, mesh=mesh,
        compiler_params=pltpu.CompilerParams(needs_layout_passes=False),
        out_type=[
            jax.ShapeDtypeStruct((BATCH * S1,), jnp.int32),
            jax.ShapeDtypeStruct((BATCH * S1 * S2,), jnp.int32),
        ],
        scratch_types=[
            pltpu.VMEM((SEEDS_W,), jnp.int32),
            pltpu.VMEM((G_W,), jnp.int32),
            pltpu.VMEM((G_W,), jnp.int32),
            pltpu.VMEM((SEEDS_W, 128), jnp.int32),
            pltpu.VMEM((E_W,), jnp.int32),
            pltpu.VMEM((E_W,), jnp.int32),
            pltpu.VMEM((CH, 128), jnp.int32),
            pltpu.VMEM((CH, 128), jnp.int32),
        ] + [pltpu.SemaphoreType.DMA] * 3,
    )()
    return f(ids, adjp, sel1, sel2)


def _gather_body(ids_h, s1_h, s2_h, p_h, q_h,
                 qs0_h, m0_h, qs1_h, m1_h,
                 ids_v, s1_v, s2_v,
                 qbuf0, qbuf1, gbuf0, gbuf1, gbuf2, gbuf3, gbuf4,
                 m1buf0, m1buf1, m1buf2, m1buf3, m1buf4,
                 m0acc_v, sem, sQ0, sQ1, sP0, sP1, sP2, sP3, sP4,
                 sW0, sW1, sM0, sM1, sM2, sM3, sM4):
    qbuf = (qbuf0, qbuf1)
    gbuf = (gbuf0, gbuf1, gbuf2, gbuf3, gbuf4)
    m1buf = (m1buf0, m1buf1, m1buf2, m1buf3, m1buf4)
    sQ, sW = (sQ0, sQ1), (sW0, sW1)
    sP, sM = (sP0, sP1, sP2, sP3, sP4), (sM0, sM1, sM2, sM3, sM4)
    w = lax.axis_index("c") * 16 + lax.axis_index("s")
    seed0 = w * SEEDS_W
    g0 = w * G_W
    e0 = w * E_W

    pltpu.sync_copy(s2_h.at[pl.ds(e0, E_W)], s2_v)
    pltpu.sync_copy(ids_h.at[pl.ds(seed0, SEEDS_W)], ids_v)
    pltpu.sync_copy(s1_h.at[pl.ds(g0, G_W)], s1_v)

    pltpu.async_copy(q_h.at[ids_v], gbuf[0].at[pl.ds(0, SEEDS_W)], sem).wait()
    pltpu.sync_copy(gbuf[0].at[pl.ds(0, SEEDS_W)], qs0_h.at[pl.ds(seed0, SEEDS_W)])

    zeros16 = jnp.zeros((16,), jnp.float32)
    def z_body(i, carry):
        for q in range(HID // 16):
            m0acc_v[i, pl.ds(q * 16, 16)] = zeros16
        return carry
    lax.fori_loop(0, SEEDS_W, z_body, 0)

    def q_gather(c, x):
        return pltpu.make_async_copy(
            q_h.at[s1_v.at[pl.ds(c * CH, CH)]], qbuf[x], sQ[x])

    def p1_gather(c, x):
        return pltpu.make_async_copy(
            p_h.at[s1_v.at[pl.ds(c * CH, CH)]], gbuf[x], sP[x])

    def qs1_write(c, x):
        return pltpu.make_async_copy(qbuf[x], qs1_h.at[pl.ds(g0 + c * CH, CH)],
                                     sW[x])

    def step1(c, x):
        q_gather(c, x).wait()
        qs1_write(c, x).start()

        p1_gather(c, x).wait()
        def acc_body(r, carry2):
            sd = (c * CH + r) // S1
            for q in range(HID // 16):
                plsc.addupdate(m0acc_v.at[sd, pl.ds(q * 16, 16)],
                               gbuf[x][r, pl.ds(q * 16, 16)])
            return carry2
        lax.fori_loop(0, CH, acc_body, 0)

        @pl.when(c + 2 < N_CH1)
        def _():
            qs1_write(c, x).wait()
            q_gather(c + 2, x).start()
            p1_gather(c + 2, x).start()

    q_gather(0, 0).start()
    p1_gather(0, 0).start()
    q_gather(1, 1).start()
    p1_gather(1, 1).start()
    def pass1(k, carry):
        step1(2 * k, 0)
        step1(2 * k + 1, 1)
        return carry
    lax.fori_loop(0, N_CH1 // 2, pass1, 0)
    qs1_write(N_CH1 - 2, 0).wait()
    qs1_write(N_CH1 - 1, 1).wait()
    pltpu.sync_copy(m0acc_v, m0_h.at[pl.ds(seed0, SEEDS_W)])

    def p2_gather(c, x):
        return pltpu.make_async_copy(
            p_h.at[s2_v.at[pl.ds(c * CH, CH)]], gbuf[x], sP[x])

    def m1_write(c, x):
        return pltpu.make_async_copy(
            m1buf[x], m1_h.at[pl.ds(g0 + c * GPC, GPC)], sM[x])

    def step2(c, x):
        p2_gather(c, x).wait()

        @pl.when(c >= NBUF)
        def _():
            m1_write(c - NBUF, x).wait()

        def grp_body(h, carry2):
            for u in range(2):
                g = h * 2 + u
                for q in range(HID // 16):
                    acc = gbuf[x][g * S2, pl.ds(q * 16, 16)]
                    for j in range(1, S2):
                        acc = acc + gbuf[x][g * S2 + j, pl.ds(q * 16, 16)]
                    m1buf[x][g, pl.ds(q * 16, 16)] = acc
            return carry2
        lax.fori_loop(0, GPC // 2, grp_body, 0)

        m1_write(c, x).start()

        @pl.when(c + NBUF < N_CH2)
        def _():
            p2_gather(c + NBUF, x).start()

    for x in range(NBUF):
        p2_gather(x, x).start()
    def pass2(k, carry):
        for x in range(NBUF):
            step2(NBUF * k + x, x)
        return carry
    lax.fori_loop(0, N_CH2 // NBUF, pass2, 0)
    for x in range(NBUF):
        m1_write(N_CH2 - NBUF + x, x).wait()


def _sc_gather(ids, s1, s2, P, Q):
    mesh = plsc.VectorSubcoreMesh(core_axis_name="c", subcore_axis_name="s",
                                  num_cores=2, num_subcores=16)
    f = functools.partial(
        pl.kernel, _gather_body, mesh=mesh,
        compiler_params=pltpu.CompilerParams(needs_layout_passes=False),
        out_type=[
            jax.ShapeDtypeStruct((BATCH, HID), jnp.float32),
            jax.ShapeDtypeStruct((BATCH, HID), jnp.float32),
            jax.ShapeDtypeStruct((BATCH * S1, HID), jnp.float32),
            jax.ShapeDtypeStruct((BATCH * S1, HID), jnp.float32),
        ],
        scratch_types=[
            pltpu.VMEM((SEEDS_W,), jnp.int32),
            pltpu.VMEM((G_W,), jnp.int32),
            pltpu.VMEM((E_W,), jnp.int32),
            pltpu.VMEM((CH, HID), jnp.float32),
            pltpu.VMEM((CH, HID), jnp.float32),
            pltpu.VMEM((CH, HID), jnp.float32),
            pltpu.VMEM((CH, HID), jnp.float32),
            pltpu.VMEM((CH, HID), jnp.float32),
            pltpu.VMEM((CH, HID), jnp.float32),
            pltpu.VMEM((CH, HID), jnp.float32),
            pltpu.VMEM((GPC, HID), jnp.float32),
            pltpu.VMEM((GPC, HID), jnp.float32),
            pltpu.VMEM((GPC, HID), jnp.float32),
            pltpu.VMEM((GPC, HID), jnp.float32),
            pltpu.VMEM((GPC, HID), jnp.float32),
            pltpu.VMEM((SEEDS_W, HID), jnp.float32),
        ] + [pltpu.SemaphoreType.DMA] * 15,
    )()
    return f(ids, s1, s2, P, Q)


def _head_body(qs1_ref, m1_ref, qs0_ref, m0_ref, a_ref,
               w2xa_ref, w2xb_ref, w2na_ref, w2nb_ref, wfca_ref, wfcb_ref,
               b1x_ref, b1n_ref, b2x_ref, b2n_ref, bfc_ref, out_ref):
    b1x = b1x_ref[...]
    b1n = b1n_ref[...]
    u = jax.nn.relu(qs1_ref[...] + b1x)
    v = jax.nn.relu(m1_ref[...] * (1.0 / S2) + b1n)
    am = a_ref[...]
    mhA = jnp.dot(am, u, preferred_element_type=jnp.float32, precision=_PREC)
    mhB = jnp.dot(am, v, preferred_element_type=jnp.float32, precision=_PREC)
    h0A = jax.nn.relu(qs0_ref[...] + b1x)
    h0B = jax.nn.relu(m0_ref[...] * (1.0 / S1) + b1n)
    gA = jax.nn.relu(
        jnp.dot(h0A, w2xa_ref[...], preferred_element_type=jnp.float32,
                precision=_PREC)
        + jnp.dot(h0B, w2xb_ref[...], preferred_element_type=jnp.float32,
                  precision=_PREC) + b2x_ref[...])
    gB = jax.nn.relu(
        jnp.dot(mhA, w2na_ref[...], preferred_element_type=jnp.float32,
                precision=_PREC)
        + jnp.dot(mhB, w2nb_ref[...], preferred_element_type=jnp.float32,
                  precision=_PREC) + b2n_ref[...])
    out_ref[...] = (
        jnp.dot(gA, wfca_ref[...], preferred_element_type=jnp.float32,
                precision=_PREC)
        + jnp.dot(gB, wfcb_ref[...], preferred_element_type=jnp.float32,
                  precision=_PREC) + bfc_ref[...])


def _head(qs1, m1, qs0, m0, A, W2x, W2n, WfcP, b1x, b1n, b2x, b2n, bfcP):
    sb = 128
    rb = sb * S1
    full = lambda i: (0, 0)
    row = lambda i: (i, 0)
    wspec = pl.BlockSpec((HID, HID), full)
    bspec = pl.BlockSpec((1, HID), full)
    return pl.pallas_call(
        _head_body,
        grid=(BATCH // sb,),
        in_specs=[
            pl.BlockSpec((rb, HID), row),
            pl.BlockSpec((rb, HID), row),
            pl.BlockSpec((sb, HID), row),
            pl.BlockSpec((sb, HID), row),
            pl.BlockSpec((sb, rb), full),
            wspec, wspec, wspec, wspec, wspec, wspec,
            bspec, bspec, bspec, bspec, bspec,
        ],
        out_specs=pl.BlockSpec((sb, HID), row),
        out_shape=jax.ShapeDtypeStruct((BATCH, HID), jnp.float32),
    )(qs1, m1, qs0, m0, A,
      W2x[:HID], W2x[HID:], W2n[:HID], W2n[HID:], WfcP[:HID], WfcP[HID:],
      b1x, b1n, b2x, b2n, bfcP)


_CONSTS = None


def _sample_idx():
    skey = jax.random.key(42)
    idx1 = jax.random.randint(jax.random.fold_in(skey, 0), (BATCH, S1), 0,
                              MAX_DEG)
    idx2 = jax.random.randint(jax.random.fold_in(skey, 1), (BATCH * S1, S2), 0,
                              MAX_DEG)
    return idx1, idx2


def _sel_tables(xp, idx1, idx2):
    seedloc = (xp.arange(BATCH, dtype=xp.int32) % SEEDS_W) * 128
    sel1 = (seedloc[:, None] + idx1).astype(xp.int32).reshape(-1)
    gloc = (xp.arange(BATCH * S1, dtype=xp.int32) % CH) * 128
    sel2 = (gloc[:, None] + idx2).astype(xp.int32).reshape(-1)
    return sel1, sel2


def _np_mean_matrix():
    import numpy as np
    return (np.repeat(np.eye(128, dtype=np.float32), S1, axis=1)
            * np.float32(1.0 / S1))


def _get_consts():
    global _CONSTS
    if _CONSTS is None:
        import numpy as np
        A = _np_mean_matrix()
        try:
            with jax.default_device(jax.devices("cpu")[0]), \
                 jax.ensure_compile_time_eval():
                idx1, idx2 = _sample_idx()
                idx1, idx2 = np.asarray(idx1), np.asarray(idx2)
        except Exception:
            sel1, sel2 = _sel_tables(jnp, *_sample_idx())
            return sel1, sel2, A
        sel1, sel2 = _sel_tables(np, idx1, idx2)
        _CONSTS = (sel1, sel2, A)
    return _CONSTS


def kernel(ids, features, adj, W1x, b1x, W1n, b1n, W2x, b2x, W2n, b2n, Wfc, bfc):
    sel1, sel2, A = _get_consts()

    ids32 = ids.astype(jnp.int32)
    adjp = jnp.pad(adj.astype(jnp.int32), ((0, 0), (0, 128 - MAX_DEG)))
    s1, s2 = _sampler(ids32, adjp, jnp.asarray(sel1), jnp.asarray(sel2))
    P, Q = _project_tables(features, W1n, W1x)
    qs0, m0, qs1, m1 = _sc_gather(ids32, s1, s2, P, Q)

    WfcP = jnp.pad(Wfc, ((0, 0), (0, HID - NUM_CLASSES)))
    bfcP = jnp.pad(bfc, (0, HID - NUM_CLASSES)).reshape(1, HID)
    out = _head(qs1, m1, qs0, m0, jnp.asarray(A), W2x, W2n, WfcP,
                b1x.reshape(1, HID), b1n.reshape(1, HID),
                b2x.reshape(1, HID), b2n.reshape(1, HID), bfcP)
    return out[:, :NUM_CLASSES]


# speedup vs baseline: 1.3205x; 1.3205x over previous
import functools

import jax
import jax.numpy as jnp
from jax import lax
from jax.experimental import pallas as pl
from jax.experimental.pallas import tpu as pltpu
import jax.experimental.pallas.tpu_sc as plsc

N_NODES = 10000
MAX_DEG = 16
D_FEAT = 256
BATCH = 1024
S1 = 25
S2 = 10
HID = 128
NUM_CLASSES = 40

NW = 32
SEEDS_W = BATCH // NW
G_W = SEEDS_W * S1
E_W = G_W * S2
CH = 80
N_CH1 = G_W // CH
N_CH2 = E_W // CH
GPC = CH // S2
NBUF = 4

_PREC = lax.Precision.DEFAULT


def _proj_body(x_ref, w1n_ref, w1x_ref, p_ref, q_ref):
    x = x_ref[...]
    p_ref[...] = jnp.dot(x, w1n_ref[...], preferred_element_type=jnp.float32,
                         precision=_PREC)
    q_ref[...] = jnp.dot(x, w1x_ref[...], preferred_element_type=jnp.float32,
                         precision=_PREC)


def _project_tables(features, W1n, W1x):
    blk = 1000
    return pl.pallas_call(
        _proj_body,
        grid=(N_NODES // blk,),
        in_specs=[
            pl.BlockSpec((blk, D_FEAT), lambda i: (i, 0)),
            pl.BlockSpec((D_FEAT, HID), lambda i: (0, 0)),
            pl.BlockSpec((D_FEAT, HID), lambda i: (0, 0)),
        ],
        out_specs=[
            pl.BlockSpec((blk, HID), lambda i: (i, 0)),
            pl.BlockSpec((blk, HID), lambda i: (i, 0)),
        ],
        out_shape=[
            jax.ShapeDtypeStruct((N_NODES, HID), jnp.float32),
            jax.ShapeDtypeStruct((N_NODES, HID), jnp.float32),
        ],
    )(features, W1n, W1x)


def _sampler_body(ids_h, adjp_h, sel1_h, sel2_h,
                  s1_h, s2_h,
                  ids_v, sel1_v, s1_v, adjseed_v, sel2_v, s2_v,
                  abuf0, abuf1, sem, sA0, sA1):
    abuf, sA = (abuf0, abuf1), (sA0, sA1)
    w = lax.axis_index("c") * 16 + lax.axis_index("s")
    seed0 = w * SEEDS_W
    g0 = w * G_W
    e0 = w * E_W

    pltpu.sync_copy(sel2_h.at[pl.ds(e0, E_W)], sel2_v)
    pltpu.sync_copy(ids_h.at[pl.ds(seed0, SEEDS_W)], ids_v)
    pltpu.sync_copy(sel1_h.at[pl.ds(g0, G_W)], sel1_v)
    pltpu.async_copy(adjp_h.at[ids_v], adjseed_v, sem).wait()

    def s1_body(k, carry):
        sel = sel1_v[pl.ds(k * 16, 16)]
        row = jnp.right_shift(sel, 7)
        col = jnp.bitwise_and(sel, 127)
        s1_v[pl.ds(k * 16, 16)] = plsc.load_gather(adjseed_v, [row, col])
        return carry
    lax.fori_loop(0, G_W // 16, s1_body, 0)
    pltpu.sync_copy(s1_v, s1_h.at[pl.ds(g0, G_W)])

    def adj_gather(c, x):
        return pltpu.make_async_copy(
            adjp_h.at[s1_v.at[pl.ds(c * CH, CH)]], abuf[x], sA[x])

    def selT(c, x):
        adj_gather(c, x).wait()
        def sel_body(k, carry):
            sel = sel2_v[pl.ds(c * (CH * S2) + k * 16, 16)]
            row = jnp.right_shift(sel, 7)
            col = jnp.bitwise_and(sel, 127)
            s2_v[pl.ds(c * (CH * S2) + k * 16, 16)] = (
                plsc.load_gather(abuf[x], [row, col]))
            return carry
        lax.fori_loop(0, CH * S2 // 16, sel_body, 0)

        @pl.when(c + 2 < N_CH1)
        def _():
            adj_gather(c + 2, x).start()

    adj_gather(0, 0).start()
    adj_gather(1, 1).start()
    def passT(k, carry):
        selT(2 * k, 0)
        selT(2 * k + 1, 1)
        return carry
    lax.fori_loop(0, N_CH1 // 2, passT, 0)
    pltpu.sync_copy(s2_v, s2_h.at[pl.ds(e0, E_W)])


def _sampler(ids, adjp, sel1, sel2):
    mesh = plsc.VectorSubcoreMesh(core_axis_name="c", subcore_axis_name="s",
                                  num_cores=2, num_subcores=16)
    f = functools.partial(
        pl.kernel, _sampler_body, mesh=mesh,
        compiler_params=pltpu.CompilerParams(needs_layout_passes=False),
        out_type=[
            jax.ShapeDtypeStruct((BATCH * S1,), jnp.int32),
            jax.ShapeDtypeStruct((BATCH * S1 * S2,), jnp.int32),
        ],
        scratch_types=[
            pltpu.VMEM((SEEDS_W,), jnp.int32),
            pltpu.VMEM((G_W,), jnp.int32),
            pltpu.VMEM((G_W,), jnp.int32),
            pltpu.VMEM((SEEDS_W, 128), jnp.int32),
            pltpu.VMEM((E_W,), jnp.int32),
            pltpu.VMEM((E_W,), jnp.int32),
            pltpu.VMEM((CH, 128), jnp.int32),
            pltpu.VMEM((CH, 128), jnp.int32),
        ] + [pltpu.SemaphoreType.DMA] * 3,
    )()
    return f(ids, adjp, sel1, sel2)


def _gather_body(ids_h, s1_h, s2_h, p_h, q_h,
                 qs0_h, m0_h, qs1_h, m1_h,
                 ids_v, s1_v, s2_v,
                 qbuf0, qbuf1, gbuf0, gbuf1, gbuf2, gbuf3,
                 m1buf0, m1buf1, m1buf2, m1buf3,
                 m0acc_v, sem, sQ0, sQ1, sP0, sP1, sP2, sP3,
                 sW0, sW1, sM0, sM1, sM2, sM3):
    qbuf = (qbuf0, qbuf1)
    gbuf, m1buf = (gbuf0, gbuf1, gbuf2, gbuf3), (m1buf0, m1buf1, m1buf2, m1buf3)
    sQ, sW = (sQ0, sQ1), (sW0, sW1)
    sP, sM = (sP0, sP1, sP2, sP3), (sM0, sM1, sM2, sM3)
    w = lax.axis_index("c") * 16 + lax.axis_index("s")
    seed0 = w * SEEDS_W
    g0 = w * G_W
    e0 = w * E_W

    pltpu.sync_copy(s2_h.at[pl.ds(e0, E_W)], s2_v)
    pltpu.sync_copy(ids_h.at[pl.ds(seed0, SEEDS_W)], ids_v)
    pltpu.sync_copy(s1_h.at[pl.ds(g0, G_W)], s1_v)

    pltpu.async_copy(q_h.at[ids_v], gbuf[0].at[pl.ds(0, SEEDS_W)], sem).wait()
    pltpu.sync_copy(gbuf[0].at[pl.ds(0, SEEDS_W)], qs0_h.at[pl.ds(seed0, SEEDS_W)])

    zeros16 = jnp.zeros((16,), jnp.float32)
    def z_body(i, carry):
        for q in range(HID // 16):
            m0acc_v[i, pl.ds(q * 16, 16)] = zeros16
        return carry
    lax.fori_loop(0, SEEDS_W, z_body, 0)

    def q_gather(c, x):
        return pltpu.make_async_copy(
            q_h.at[s1_v.at[pl.ds(c * CH, CH)]], qbuf[x], sQ[x])

    def p1_gather(c, x):
        return pltpu.make_async_copy(
            p_h.at[s1_v.at[pl.ds(c * CH, CH)]], gbuf[x], sP[x])

    def qs1_write(c, x):
        return pltpu.make_async_copy(qbuf[x], qs1_h.at[pl.ds(g0 + c * CH, CH)],
                                     sW[x])

    def step1(c, x):
        q_gather(c, x).wait()
        qs1_write(c, x).start()

        p1_gather(c, x).wait()
        def acc_body(r, carry2):
            sd = (c * CH + r) // S1
            for q in range(HID // 16):
                plsc.addupdate(m0acc_v.at[sd, pl.ds(q * 16, 16)],
                               gbuf[x][r, pl.ds(q * 16, 16)])
            return carry2
        lax.fori_loop(0, CH, acc_body, 0)

        @pl.when(c + 2 < N_CH1)
        def _():
            qs1_write(c, x).wait()
            q_gather(c + 2, x).start()
            p1_gather(c + 2, x).start()

    q_gather(0, 0).start()
    p1_gather(0, 0).start()
    q_gather(1, 1).start()
    p1_gather(1, 1).start()
    def pass1(k, carry):
        step1(2 * k, 0)
        step1(2 * k + 1, 1)
        return carry
    lax.fori_loop(0, N_CH1 // 2, pass1, 0)
    qs1_write(N_CH1 - 2, 0).wait()
    qs1_write(N_CH1 - 1, 1).wait()
    pltpu.sync_copy(m0acc_v, m0_h.at[pl.ds(seed0, SEEDS_W)])

    def p2_gather(c, x):
        return pltpu.make_async_copy(
            p_h.at[s2_v.at[pl.ds(c * CH, CH)]], gbuf[x], sP[x])

    def m1_write(c, x):
        return pltpu.make_async_copy(
            m1buf[x], m1_h.at[pl.ds(g0 + c * GPC, GPC)], sM[x])

    def step2(c, x):
        p2_gather(c, x).wait()

        @pl.when(c >= NBUF)
        def _():
            m1_write(c - NBUF, x).wait()

        def grp_body(g, carry2):
            for q in range(HID // 16):
                acc = gbuf[x][g * S2, pl.ds(q * 16, 16)]
                for j in range(1, S2):
                    acc = acc + gbuf[x][g * S2 + j, pl.ds(q * 16, 16)]
                m1buf[x][g, pl.ds(q * 16, 16)] = acc
            return carry2
        lax.fori_loop(0, GPC, grp_body, 0)

        m1_write(c, x).start()

        @pl.when(c + NBUF < N_CH2)
        def _():
            p2_gather(c + NBUF, x).start()

    for x in range(NBUF):
        p2_gather(x, x).start()
    def pass2(k, carry):
        for x in range(NBUF):
            step2(NBUF * k + x, x)
        return carry
    lax.fori_loop(0, N_CH2 // NBUF, pass2, 0)
    for x in range(NBUF):
        m1_write(N_CH2 - NBUF + x, x).wait()


def _sc_gather(ids, s1, s2, P, Q):
    mesh = plsc.VectorSubcoreMesh(core_axis_name="c", subcore_axis_name="s",
                                  num_cores=2, num_subcores=16)
    f = functools.partial(
        pl.kernel, _gather_body, mesh=mesh,
        compiler_params=pltpu.CompilerParams(needs_layout_passes=False),
        out_type=[
            jax.ShapeDtypeStruct((BATCH, HID), jnp.float32),
            jax.ShapeDtypeStruct((BATCH, HID), jnp.float32),
            jax.ShapeDtypeStruct((BATCH * S1, HID), jnp.float32),
            jax.ShapeDtypeStruct((BATCH * S1, HID), jnp.float32),
        ],
        scratch_types=[
            pltpu.VMEM((SEEDS_W,), jnp.int32),
            pltpu.VMEM((G_W,), jnp.int32),
            pltpu.VMEM((E_W,), jnp.int32),
            pltpu.VMEM((CH, HID), jnp.float32),
            pltpu.VMEM((CH, HID), jnp.float32),
            pltpu.VMEM((CH, HID), jnp.float32),
            pltpu.VMEM((CH, HID), jnp.float32),
            pltpu.VMEM((CH, HID), jnp.float32),
            pltpu.VMEM((CH, HID), jnp.float32),
            pltpu.VMEM((GPC, HID), jnp.float32),
            pltpu.VMEM((GPC, HID), jnp.float32),
            pltpu.VMEM((GPC, HID), jnp.float32),
            pltpu.VMEM((GPC, HID), jnp.float32),
            pltpu.VMEM((SEEDS_W, HID), jnp.float32),
        ] + [pltpu.SemaphoreType.DMA] * 13,
    )()
    return f(ids, s1, s2, P, Q)


def _head_body(qs1_ref, m1_ref, qs0_ref, m0_ref, a_ref,
               w2xa_ref, w2xb_ref, w2na_ref, w2nb_ref, wfca_ref, wfcb_ref,
               b1x_ref, b1n_ref, b2x_ref, b2n_ref, bfc_ref, out_ref):
    b1x = b1x_ref[...]
    b1n = b1n_ref[...]
    u = jax.nn.relu(qs1_ref[...] + b1x)
    v = jax.nn.relu(m1_ref[...] * (1.0 / S2) + b1n)
    am = a_ref[...]
    mhA = jnp.dot(am, u, preferred_element_type=jnp.float32, precision=_PREC)
    mhB = jnp.dot(am, v, preferred_element_type=jnp.float32, precision=_PREC)
    h0A = jax.nn.relu(qs0_ref[...] + b1x)
    h0B = jax.nn.relu(m0_ref[...] * (1.0 / S1) + b1n)
    gA = jax.nn.relu(
        jnp.dot(h0A, w2xa_ref[...], preferred_element_type=jnp.float32,
                precision=_PREC)
        + jnp.dot(h0B, w2xb_ref[...], preferred_element_type=jnp.float32,
                  precision=_PREC) + b2x_ref[...])
    gB = jax.nn.relu(
        jnp.dot(mhA, w2na_ref[...], preferred_element_type=jnp.float32,
                precision=_PREC)
        + jnp.dot(mhB, w2nb_ref[...], preferred_element_type=jnp.float32,
                  precision=_PREC) + b2n_ref[...])
    out_ref[...] = (
        jnp.dot(gA, wfca_ref[...], preferred_element_type=jnp.float32,
                precision=_PREC)
        + jnp.dot(gB, wfcb_ref[...], preferred_element_type=jnp.float32,
                  precision=_PREC) + bfc_ref[...])


def _head(qs1, m1, qs0, m0, A, W2x, W2n, WfcP, b1x, b1n, b2x, b2n, bfcP):
    sb = 128
    rb = sb * S1
    full = lambda i: (0, 0)
    row = lambda i: (i, 0)
    wspec = pl.BlockSpec((HID, HID), full)
    bspec = pl.BlockSpec((1, HID), full)
    return pl.pallas_call(
        _head_body,
        grid=(BATCH // sb,),
        in_specs=[
            pl.BlockSpec((rb, HID), row),
            pl.BlockSpec((rb, HID), row),
            pl.BlockSpec((sb, HID), row),
            pl.BlockSpec((sb, HID), row),
            pl.BlockSpec((sb, rb), full),
            wspec, wspec, wspec, wspec, wspec, wspec,
            bspec, bspec, bspec, bspec, bspec,
        ],
        out_specs=pl.BlockSpec((sb, HID), row),
        out_shape=jax.ShapeDtypeStruct((BATCH, HID), jnp.float32),
    )(qs1, m1, qs0, m0, A,
      W2x[:HID], W2x[HID:], W2n[:HID], W2n[HID:], WfcP[:HID], WfcP[HID:],
      b1x, b1n, b2x, b2n, bfcP)


_CONSTS = None


def _sample_idx():
    skey = jax.random.key(42)
    idx1 = jax.random.randint(jax.random.fold_in(skey, 0), (BATCH, S1), 0,
                              MAX_DEG)
    idx2 = jax.random.randint(jax.random.fold_in(skey, 1), (BATCH * S1, S2), 0,
                              MAX_DEG)
    return idx1, idx2


def _sel_tables(xp, idx1, idx2):
    seedloc = (xp.arange(BATCH, dtype=xp.int32) % SEEDS_W) * 128
    sel1 = (seedloc[:, None] + idx1).astype(xp.int32).reshape(-1)
    gloc = (xp.arange(BATCH * S1, dtype=xp.int32) % CH) * 128
    sel2 = (gloc[:, None] + idx2).astype(xp.int32).reshape(-1)
    return sel1, sel2


def _np_mean_matrix():
    import numpy as np
    return (np.repeat(np.eye(128, dtype=np.float32), S1, axis=1)
            * np.float32(1.0 / S1))


def _get_consts():
    global _CONSTS
    if _CONSTS is None:
        import numpy as np
        A = _np_mean_matrix()
        try:
            with jax.default_device(jax.devices("cpu")[0]), \
                 jax.ensure_compile_time_eval():
                idx1, idx2 = _sample_idx()
                idx1, idx2 = np.asarray(idx1), np.asarray(idx2)
        except Exception:
            sel1, sel2 = _sel_tables(jnp, *_sample_idx())
            return sel1, sel2, A
        sel1, sel2 = _sel_tables(np, idx1, idx2)
        _CONSTS = (sel1, sel2, A)
    return _CONSTS


def kernel(ids, features, adj, W1x, b1x, W1n, b1n, W2x, b2x, W2n, b2n, Wfc, bfc):
    sel1, sel2, A = _get_consts()

    ids32 = ids.astype(jnp.int32)
    adjp = jnp.pad(adj.astype(jnp.int32), ((0, 0), (0, 128 - MAX_DEG)))
    s1, s2 = _sampler(ids32, adjp, jnp.asarray(sel1), jnp.asarray(sel2))
    P, Q = _project_tables(features, W1n, W1x)
    qs0, m0, qs1, m1 = _sc_gather(ids32, s1, s2, P, Q)

    WfcP = jnp.pad(Wfc, ((0, 0), (0, HID - NUM_CLASSES)))
    bfcP = jnp.pad(bfc, (0, HID - NUM_CLASSES)).reshape(1, HID)
    out = _head(qs1, m1, qs0, m0, jnp.asarray(A), W2x, W2n, WfcP,
                b1x.reshape(1, HID), b1n.reshape(1, HID),
                b2x.reshape(1, HID), b2n.reshape(1, HID), bfcP)
    return out[:, :NUM_CLASSES]
